# Initial kernel scaffold; baseline (speedup 1.0000x reference)
import jax
import jax.numpy as jnp
from jax.experimental import pallas as pl


def kernel(hidden_states, gate_w, e_bias, w13, w2, shared_w13, shared_w2):
    raise NotImplementedError("write your pallas kernel here")


# speedup vs baseline: 1.1389x; 1.1389x over previous
import functools

import jax
import jax.numpy as jnp
from jax import lax
from jax.experimental import pallas as pl
from jax.experimental.pallas import tpu as pltpu

_T = 2048
_H = 1024
_E = 16
_K = 2
_I = 512
_NG = 2
_GS = _E // _NG
_RSF = 2.5
_NEG = -1e30


def _router_combine(x, gate_w, e_bias):
    logits = jnp.dot(x, gate_w.T, preferred_element_type=jnp.float32)
    s = jax.nn.sigmoid(logits)
    sb = s + e_bias
    lane = lax.broadcasted_iota(jnp.int32, logits.shape, 1)
    gid = lane // _GS

    def top2_masked(v):
        m1 = jnp.max(v, axis=1, keepdims=True)
        i1 = jnp.min(jnp.where(v == m1, lane, _E + 1), axis=1, keepdims=True)
        v2 = jnp.where(lane == i1, _NEG, v)
        m2 = jnp.max(v2, axis=1, keepdims=True)
        return m1, i1, m2

    sb0 = jnp.where(gid == 0, sb, _NEG)
    sb1 = jnp.where(gid == 1, sb, _NEG)
    m1a, _, m2a = top2_masked(sb0)
    m1b, _, m2b = top2_masked(sb1)
    g0 = m1a + m2a
    g1 = m1b + m2b
    chosen = jnp.where(g0 >= g1, 0, 1)
    masked = jnp.where(gid == chosen, sb, _NEG)
    _, e1, _ = top2_masked(masked)
    masked2 = jnp.where(lane == e1, _NEG, masked)
    m1c = jnp.max(masked2, axis=1, keepdims=True)
    e2 = jnp.min(jnp.where(masked2 == m1c, lane, _E + 1), axis=1, keepdims=True)
    w1 = jnp.sum(jnp.where(lane == e1, s, 0.0), axis=1, keepdims=True)
    w2 = jnp.sum(jnp.where(lane == e2, s, 0.0), axis=1, keepdims=True)
    wn = w1 + w2 + 1e-20
    combine = (lane == e1) * (w1 / wn) + (lane == e2) * (w2 / wn)
    return combine.astype(jnp.float32) * _RSF


_CT = 256


def _moe_body(x_ref, gate_ref, bias_ref, w13_ref, w2_ref, sw13_ref, sw2_ref,
              out_ref, comb_ref):
    e = pl.program_id(0)
    lane = lax.broadcasted_iota(jnp.int32, (_CT, _E), 1)

    def chunk(i, carry):
        sl = pl.ds(i * _CT, _CT)
        xc = x_ref[sl, :]

        @pl.when(e == 0)
        def _init():
            comb_ref[sl, :] = _router_combine(xc, gate_ref[...], bias_ref[...])
            sgu = jnp.dot(xc, sw13_ref[...].T, preferred_element_type=jnp.float32)
            sact = jax.nn.silu(sgu[:, :_I]) * sgu[:, _I:]
            out_ref[sl, :] = jnp.dot(sact, sw2_ref[...].T,
                                     preferred_element_type=jnp.float32)

        gu = jnp.dot(xc, w13_ref[0].T, preferred_element_type=jnp.float32)
        act = jax.nn.silu(gu[:, :_I]) * gu[:, _I:]
        col = jnp.sum(jnp.where(lane == e, comb_ref[sl, :], 0.0),
                      axis=1, keepdims=True)
        out_ref[sl, :] += jnp.dot(act * col, w2_ref[0].T,
                                  preferred_element_type=jnp.float32)
        return carry

    lax.fori_loop(0, _T // _CT, chunk, 0)


@jax.jit
def kernel(hidden_states, gate_w, e_bias, w13, w2, shared_w13, shared_w2):
    grid_spec = pltpu.PrefetchScalarGridSpec(
        num_scalar_prefetch=0,
        grid=(_E,),
        in_specs=[
            pl.BlockSpec((_T, _H), lambda e: (0, 0)),
            pl.BlockSpec((_E, _H), lambda e: (0, 0)),
            pl.BlockSpec((1, _E), lambda e: (0, 0)),
            pl.BlockSpec((1, 2 * _I, _H), lambda e: (e, 0, 0)),
            pl.BlockSpec((1, _H, _I), lambda e: (e, 0, 0)),
            pl.BlockSpec((2 * _I, _H), lambda e: (0, 0)),
            pl.BlockSpec((_H, _I), lambda e: (0, 0)),
        ],
        out_specs=pl.BlockSpec((_T, _H), lambda e: (0, 0)),
        scratch_shapes=[
            pltpu.VMEM((_T, _E), jnp.float32),
        ],
    )
    return pl.pallas_call(
        _moe_body,
        grid_spec=grid_spec,
        out_shape=jax.ShapeDtypeStruct((_T, _H), jnp.float32),
        compiler_params=pltpu.CompilerParams(
            dimension_semantics=("arbitrary",),
        ),
    )(hidden_states, gate_w, e_bias.reshape(1, _E), w13, w2, shared_w13, shared_w2)
